# baseline (speedup 1.0000x reference)
import jax
import jax.numpy as jnp
from jax import lax
from jax.experimental import pallas as pl
from jax.experimental.pallas import tpu as pltpu
from jax.experimental.pallas import tpu_sc as plsc

P = 256
N = 16
K = 64
NK = N * K
MP = 256
D = 256
L = 16
NTILES = 32
PPT = P // NTILES
CW = 384
NQ = 2 * PPT


def _body(feats_hbm, seed_hbm, nb_hbm, comb_hbm,
          outf_hbm, outp_hbm, outm_hbm,
          seed8, nb8, combP0, combP1, pos_v,
          fidxP0, fidxP1, msk8, pts8,
          ring0, ring1, ring2,
          semC0, semC1, semG0, semG1, semG2, semO0, semO1, semO2):
    wid = lax.axis_index("c") * 16 + lax.axis_index("s")
    iota = lax.iota(jnp.int32, L)
    zeros_f = jnp.zeros((L,), jnp.float32)
    base_p = wid * PPT

    combP = (combP0, combP1)
    fidxP = (fidxP0, fidxP1)
    ring = (ring0, ring1, ring2)
    semC = (semC0, semC1)
    semG = (semG0, semG1, semG2)
    semO = (semO0, semO1, semO2)

    pltpu.sync_copy(seed_hbm.at[pl.ds(base_p * N, PPT * N)], seed8)
    pltpu.sync_copy(nb_hbm.at[pl.ds(base_p * N, PPT * N)], nb8)
    comb_descs = [None, None]
    comb_descs[0] = pltpu.async_copy(
        comb_hbm.at[seed8.at[pl.ds(0, N)]], combP[0], semC[0])

    gather_descs = [None] * NQ
    out_descs = [None] * NQ
    cmin_hist = [None] * PPT

    def finish_chunk(i, h):
        qq = 2 * i + h
        gather_descs[qq].wait()
        srow = jnp.clip(cmin_hist[i] - h * 128, 0, 128)
        rbuf = ring[qq % 3]

        def zero_row(row, _, rbuf=rbuf):
            for w in range(D // L):
                rbuf[row, pl.ds(w * L, L)] = zeros_f
            return 0

        lax.fori_loop(srow, 128, zero_row, 0)
        out_descs[qq] = pltpu.async_copy(
            rbuf, outf_hbm.at[pl.ds((base_p + i) * MP + h * 128, 128)],
            semO[qq % 3])

    for i in range(PPT):
        par = i & 1
        cb = combP[par]
        comb_descs[par].wait()
        if i + 1 < PPT:
            comb_descs[1 - par] = pltpu.async_copy(
                comb_hbm.at[seed8.at[pl.ds((i + 1) * N, N)]],
                combP[1 - par], semC[1 - par])

        def rank_node(n, cnt_vec, cb=cb, i=i):
            nb16 = plsc.load_gather(nb8, [jnp.full((L,), i * N + n, jnp.int32)])
            cv = cnt_vec
            for t in range(4):
                ix16 = cb[n, pl.ds(t * L, L)]
                m16 = jnp.where(ix16 >= 0, 1, 0)
                v16 = m16 * nb16
                inc = plsc.cumsum(v16)
                ranks = jnp.maximum(cv + inc - 1, 0)
                cols = n * K + t * L + iota
                plsc.store_scatter(pos_v, [ranks], cols, mask=v16 > 0)
                cv = cv + plsc.all_reduce_population_count(v16 > 0)
            return cv

        cnt_vec = lax.fori_loop(0, N, rank_node, jnp.zeros((L,), jnp.int32))
        c = jnp.sum(cnt_vec) >> 4
        cmin = jnp.minimum(c, MP)
        cmin_hist[i] = cmin

        for u in range(MP // L):
            j16 = u * L + iota
            r16 = jnp.where(c > MP, (j16 * c) >> 8, j16)
            src = plsc.load_gather(pos_v, [r16]) & (NK - 1)
            n_idx = src >> 6
            k_idx = src & (K - 1)
            valid = j16 < cmin
            fx = plsc.load_gather(cb, [n_idx, k_idx])
            fx = jnp.maximum(fx, 0)
            px = plsc.bitcast(
                plsc.load_gather(cb, [n_idx, K + 3 * k_idx]), jnp.float32)
            py = plsc.bitcast(
                plsc.load_gather(cb, [n_idx, K + 3 * k_idx + 1]), jnp.float32)
            pz = plsc.bitcast(
                plsc.load_gather(cb, [n_idx, K + 3 * k_idx + 2]), jnp.float32)
            px = jnp.where(valid, px, 0.0)
            py = jnp.where(valid, py, 0.0)
            pz = jnp.where(valid, pz, 0.0)
            fidxP[par][pl.ds(u * L, L)] = fx
            msk8[pl.ds(i * MP + u * L, L)] = valid.astype(jnp.int32)
            pts8[pl.ds(0 * PPT * MP + i * MP + u * L, L)] = px
            pts8[pl.ds(1 * PPT * MP + i * MP + u * L, L)] = py
            pts8[pl.ds(2 * PPT * MP + i * MP + u * L, L)] = pz

        if i >= 1:
            finish_chunk(i - 1, 0)
            finish_chunk(i - 1, 1)

        for h in range(2):
            q = 2 * i + h
            if q >= 3:
                out_descs[q - 3].wait()
            gather_descs[q] = pltpu.async_copy(
                feats_hbm.at[fidxP[par].at[pl.ds(h * 128, 128)]],
                ring[q % 3], semG[q % 3])

    finish_chunk(PPT - 1, 0)
    finish_chunk(PPT - 1, 1)
    for qq in range(NQ - 3, NQ):
        out_descs[qq].wait()
    pltpu.sync_copy(msk8, outm_hbm.at[pl.ds(base_p * MP, PPT * MP)])
    for cc in range(3):
        pltpu.sync_copy(
            pts8.at[pl.ds(cc * PPT * MP, PPT * MP)],
            outp_hbm.at[pl.ds(cc * P * MP + base_p * MP, PPT * MP)])


@jax.jit
def _run(feats, seed_flat, nb_flat, comb):
    kfn = pl.kernel(
        _body,
        out_type=[
            jax.ShapeDtypeStruct((P * MP, D), jnp.float32),
            jax.ShapeDtypeStruct((P * 3 * MP,), jnp.float32),
            jax.ShapeDtypeStruct((P * MP,), jnp.int32),
        ],
        mesh=plsc.VectorSubcoreMesh(core_axis_name="c", subcore_axis_name="s"),
        compiler_params=pltpu.CompilerParams(needs_layout_passes=False),
        scratch_types=[
            pltpu.VMEM((PPT * N,), jnp.int32),
            pltpu.VMEM((PPT * N,), jnp.int32),
            pltpu.VMEM((N, CW), jnp.int32),
            pltpu.VMEM((N, CW), jnp.int32),
            pltpu.VMEM((NK,), jnp.int32),
            pltpu.VMEM((MP,), jnp.int32),
            pltpu.VMEM((MP,), jnp.int32),
            pltpu.VMEM((PPT * MP,), jnp.int32),
            pltpu.VMEM((PPT * 3 * MP,), jnp.float32),
            pltpu.VMEM((128, D), jnp.float32),
            pltpu.VMEM((128, D), jnp.float32),
            pltpu.VMEM((128, D), jnp.float32),
            pltpu.SemaphoreType.DMA,
            pltpu.SemaphoreType.DMA,
            pltpu.SemaphoreType.DMA,
            pltpu.SemaphoreType.DMA,
            pltpu.SemaphoreType.DMA,
            pltpu.SemaphoreType.DMA,
            pltpu.SemaphoreType.DMA,
            pltpu.SemaphoreType.DMA,
        ],
    )
    return kfn(feats, seed_flat, nb_flat, comb)


def kernel(ref_node_neighbor_mask, ref_seed_neighbor_indices, ref_node_knn_masks,
           ref_node_knn_points, ref_node_knn_indices, ref_feats_m):
    num_nodes = ref_node_knn_masks.shape[0]
    comb = jnp.concatenate([
        jnp.where(ref_node_knn_masks, ref_node_knn_indices.astype(jnp.int32),
                  jnp.int32(-1)),
        lax.bitcast_convert_type(
            ref_node_knn_points.reshape(num_nodes, 3 * K), jnp.int32),
        jnp.zeros((num_nodes, CW - 4 * K), jnp.int32),
    ], axis=1)
    outf, outp, outm = _run(
        ref_feats_m,
        ref_seed_neighbor_indices.astype(jnp.int32).reshape(-1),
        ref_node_neighbor_mask.astype(jnp.int32).reshape(-1),
        comb)
    local_feats = outf.reshape(P, MP, D)
    local_points = outp.reshape(3, P, MP).transpose(1, 2, 0)
    local_masks = outm.reshape(P, MP).astype(bool)
    return local_points, local_feats, local_masks


# speedup vs baseline: 2.5823x; 2.5823x over previous
import jax
import jax.numpy as jnp
from jax import lax
from jax.experimental import pallas as pl
from jax.experimental.pallas import tpu as pltpu
from jax.experimental.pallas import tpu_sc as plsc

P = 256
N = 16
K = 64
NK = N * K
MP = 256
D = 256
L = 16
NTILES = 32
PPT = P // NTILES
CW = 384
NQ = 2 * PPT


def _body(feats_hbm, seed_hbm, nb_hbm, comb_hbm,
          outf_hbm, outp_hbm, outm_hbm,
          seed8, nb8, combP0, combP1, pos_v,
          fidxP0, fidxP1, msk8, pts8,
          ring0, ring1, ring2,
          semC0, semC1, semG0, semG1, semG2, semO0, semO1, semO2):
    wid = lax.axis_index("c") * 16 + lax.axis_index("s")
    iota = lax.iota(jnp.int32, L)
    zeros_f = jnp.zeros((L,), jnp.float32)
    base_p = wid * PPT

    combP = (combP0, combP1)
    fidxP = (fidxP0, fidxP1)
    ring = (ring0, ring1, ring2)
    semC = (semC0, semC1)
    semG = (semG0, semG1, semG2)
    semO = (semO0, semO1, semO2)

    pltpu.sync_copy(seed_hbm.at[pl.ds(base_p * N, PPT * N)], seed8)
    pltpu.sync_copy(nb_hbm.at[pl.ds(base_p * N, PPT * N)], nb8)
    comb_descs = [None, None]
    comb_descs[0] = pltpu.async_copy(
        comb_hbm.at[seed8.at[pl.ds(0, N)]], combP[0], semC[0])

    gather_descs = [None] * NQ
    out_descs = [None] * NQ
    cmin_hist = [None] * PPT

    def finish_chunk(i, h):
        qq = 2 * i + h
        gather_descs[qq].wait()
        srow = jnp.clip(cmin_hist[i] - h * 128, 0, 128)
        rbuf = ring[qq % 3]

        def zero_row(row, _, rbuf=rbuf):
            for w in range(D // L):
                rbuf[row, pl.ds(w * L, L)] = zeros_f
            return 0

        lax.fori_loop(srow, 128, zero_row, 0)
        out_descs[qq] = pltpu.async_copy(
            rbuf, outf_hbm.at[pl.ds((base_p + i) * MP + h * 128, 128)],
            semO[qq % 3])

    for i in range(PPT):
        par = i & 1
        cb = combP[par]
        comb_descs[par].wait()
        if i + 1 < PPT:
            comb_descs[1 - par] = pltpu.async_copy(
                comb_hbm.at[seed8.at[pl.ds((i + 1) * N, N)]],
                combP[1 - par], semC[1 - par])

        def rank_node(n, cnt_vec, cb=cb, i=i):
            nb16 = plsc.load_gather(nb8, [jnp.full((L,), i * N + n, jnp.int32)])
            cv = cnt_vec
            for t in range(4):
                m16 = cb[n, pl.ds(t * L, L)]
                v16 = m16 * nb16
                inc = plsc.cumsum(v16)
                ranks = jnp.maximum(cv + inc - 1, 0)
                cols = n * K + t * L + iota
                plsc.store_scatter(pos_v, [ranks], cols, mask=v16 > 0)
                cv = cv + plsc.all_reduce_population_count(v16 > 0)
            return cv

        cnt_vec = lax.fori_loop(0, N, rank_node, jnp.zeros((L,), jnp.int32))
        c = jnp.sum(cnt_vec) >> 4
        cmin = jnp.minimum(c, MP)
        cmin_hist[i] = cmin

        for u in range(MP // L):
            j16 = u * L + iota
            r16 = jnp.where(c > MP, (j16 * c) >> 8, j16)
            src = plsc.load_gather(pos_v, [r16]) & (NK - 1)
            n_idx = src >> 6
            k_idx = src & (K - 1)
            valid = j16 < cmin
            fx = plsc.load_gather(cb, [n_idx, K + k_idx])
            px = plsc.bitcast(
                plsc.load_gather(cb, [n_idx, 2 * K + 3 * k_idx]), jnp.float32)
            py = plsc.bitcast(
                plsc.load_gather(cb, [n_idx, 2 * K + 3 * k_idx + 1]), jnp.float32)
            pz = plsc.bitcast(
                plsc.load_gather(cb, [n_idx, 2 * K + 3 * k_idx + 2]), jnp.float32)
            px = jnp.where(valid, px, 0.0)
            py = jnp.where(valid, py, 0.0)
            pz = jnp.where(valid, pz, 0.0)
            fidxP[par][pl.ds(u * L, L)] = fx
            msk8[pl.ds(i * MP + u * L, L)] = valid.astype(jnp.int32)
            pts8[pl.ds(0 * PPT * MP + i * MP + u * L, L)] = px
            pts8[pl.ds(1 * PPT * MP + i * MP + u * L, L)] = py
            pts8[pl.ds(2 * PPT * MP + i * MP + u * L, L)] = pz

        if i >= 1:
            finish_chunk(i - 1, 0)
            finish_chunk(i - 1, 1)

        for h in range(2):
            q = 2 * i + h
            if q >= 3:
                out_descs[q - 3].wait()
            gather_descs[q] = pltpu.async_copy(
                feats_hbm.at[fidxP[par].at[pl.ds(h * 128, 128)]],
                ring[q % 3], semG[q % 3])

    finish_chunk(PPT - 1, 0)
    finish_chunk(PPT - 1, 1)
    for qq in range(NQ - 3, NQ):
        out_descs[qq].wait()
    pltpu.sync_copy(msk8, outm_hbm.at[pl.ds(base_p * MP, PPT * MP)])
    for cc in range(3):
        pltpu.sync_copy(
            pts8.at[pl.ds(cc * PPT * MP, PPT * MP)],
            outp_hbm.at[pl.ds(cc * P * MP + base_p * MP, PPT * MP)])


@jax.jit
def _run(feats, seed_flat, nb_flat, comb):
    kfn = pl.kernel(
        _body,
        out_type=[
            jax.ShapeDtypeStruct((P * MP, D), jnp.float32),
            jax.ShapeDtypeStruct((P * 3 * MP,), jnp.float32),
            jax.ShapeDtypeStruct((P * MP,), jnp.int32),
        ],
        mesh=plsc.VectorSubcoreMesh(core_axis_name="c", subcore_axis_name="s"),
        compiler_params=pltpu.CompilerParams(needs_layout_passes=False),
        scratch_types=[
            pltpu.VMEM((PPT * N,), jnp.int32),
            pltpu.VMEM((PPT * N,), jnp.int32),
            pltpu.VMEM((N, CW), jnp.int32),
            pltpu.VMEM((N, CW), jnp.int32),
            pltpu.VMEM((NK,), jnp.int32),
            pltpu.VMEM((MP,), jnp.int32),
            pltpu.VMEM((MP,), jnp.int32),
            pltpu.VMEM((PPT * MP,), jnp.int32),
            pltpu.VMEM((PPT * 3 * MP,), jnp.float32),
            pltpu.VMEM((128, D), jnp.float32),
            pltpu.VMEM((128, D), jnp.float32),
            pltpu.VMEM((128, D), jnp.float32),
            pltpu.SemaphoreType.DMA,
            pltpu.SemaphoreType.DMA,
            pltpu.SemaphoreType.DMA,
            pltpu.SemaphoreType.DMA,
            pltpu.SemaphoreType.DMA,
            pltpu.SemaphoreType.DMA,
            pltpu.SemaphoreType.DMA,
            pltpu.SemaphoreType.DMA,
        ],
    )
    return kfn(feats, seed_flat, nb_flat, comb)


def kernel(ref_node_neighbor_mask, ref_seed_neighbor_indices, ref_node_knn_masks,
           ref_node_knn_points, ref_node_knn_indices, ref_feats_m):
    num_nodes = ref_node_knn_masks.shape[0]
    comb = jnp.concatenate([
        ref_node_knn_masks.astype(jnp.int32),
        ref_node_knn_indices.astype(jnp.int32),
        lax.bitcast_convert_type(
            ref_node_knn_points.reshape(num_nodes, 3 * K), jnp.int32),
        jnp.zeros((num_nodes, CW - 5 * K), jnp.int32),
    ], axis=1)
    outf, outp, outm = _run(
        ref_feats_m,
        ref_seed_neighbor_indices.astype(jnp.int32).reshape(-1),
        ref_node_neighbor_mask.astype(jnp.int32).reshape(-1),
        comb)
    local_feats = outf.reshape(P, MP, D)
    local_points = outp.reshape(3, P, MP).transpose(1, 2, 0)
    local_masks = outm.reshape(P, MP).astype(bool)
    return local_points, local_feats, local_masks
